# baseline (speedup 1.0000x reference)
import functools

import jax
import jax.numpy as jnp
from jax import lax
from jax.experimental import pallas as pl
from jax.experimental.pallas import tpu as pltpu
from jax.experimental.pallas import tpu_sc as plsc

NC, NS = 2, 16
NW = NC * NS
LANES = 16


def _rsqrt(v):
    s = v * 0.5 + 0.5
    for _ in range(24):
        s = 0.5 * (s + v / s)
    return 1.0 / s


@functools.lru_cache(maxsize=None)
def _build(B, SEQ):
    rows_per_w = B // NW
    chunk = 128
    nchunk = rows_per_w // chunk

    mesh = plsc.VectorSubcoreMesh(
        core_axis_name="c", subcore_axis_name="s",
        num_cores=NC, num_subcores=NS)

    @functools.partial(
        pl.kernel,
        out_type=jax.ShapeDtypeStruct((B, 2), jnp.float32),
        mesh=mesh,
        scratch_types=[
            pltpu.VMEM((chunk, SEQ), jnp.int32),
            pltpu.VMEM((4, LANES), jnp.float32),
            pltpu.VMEM((32,), jnp.float32),
            pltpu.VMEM((LANES,), jnp.float32),
            pltpu.VMEM((LANES,), jnp.float32),
            pltpu.VMEM((rows_per_w, 2), jnp.float32),
        ],
        compiler_params=pltpu.CompilerParams(use_tc_tiling_on_sc=True,
                                             needs_layout_passes=False),
    )
    def tiny_kernel(x_hbm, tcols_hbm, params_hbm, out_hbm,
                    xv, tcols_v, params_v, proj0_v, proj1_v, outv):
        wid = lax.axis_index("s") * NC + lax.axis_index("c")

        pltpu.sync_copy(tcols_hbm, tcols_v)
        pltpu.sync_copy(params_hbm, params_v)

        pa = params_v[pl.ds(0, LANES)]
        pb = params_v[pl.ds(LANES, LANES)]

        c = [tcols_v[k] for k in range(4)]
        mu = (c[0] + c[1] + c[2] + c[3]) * 0.25
        d = [ck - mu for ck in c]
        var = (d[0] * d[0] + d[1] * d[1] + d[2] * d[2] + d[3] * d[3]) * 0.25
        r = _rsqrt(var + 1e-5)
        ln = [d[k] * r * pa[k] + pa[4 + k] for k in range(4)]
        inv = 1.0 / SEQ
        t0 = (ln[0] * pa[8] + ln[1] * pa[9]
              + ln[2] * pa[10] + ln[3] * pa[11]
              + pb[0]) * inv
        t1 = (ln[0] * pa[12] + ln[1] * pa[13]
              + ln[2] * pa[14] + ln[3] * pa[15]
              + pb[1]) * inv
        proj0_v[...] = t0
        proj1_v[...] = t1

        iota = lax.iota(jnp.int32, LANES)
        zeros = jnp.zeros((LANES,), jnp.float32)

        for ci in range(nchunk):
            base = wid * rows_per_w + ci * chunk
            pltpu.sync_copy(x_hbm.at[pl.ds(base, chunk)], xv)
            for g in range(chunk // LANES):
                rows = g * LANES + iota

                def lbody(l, acc, rows=rows):
                    a0, a1 = acc
                    colv = jnp.full((LANES,), l, jnp.int32)
                    xi = plsc.load_gather(xv, [rows, colv])
                    a0 = a0 + plsc.load_gather(proj0_v, [xi])
                    a1 = a1 + plsc.load_gather(proj1_v, [xi])
                    return a0, a1

                a0, a1 = lax.fori_loop(0, SEQ, lbody, (zeros, zeros),
                                       unroll=8)
                orow = ci * chunk + g * LANES + iota
                plsc.store_scatter(outv, [orow, jnp.zeros((LANES,), jnp.int32)], a0)
                plsc.store_scatter(outv, [orow, jnp.ones((LANES,), jnp.int32)], a1)

        pltpu.sync_copy(outv, out_hbm.at[pl.ds(wid * rows_per_w, rows_per_w)])

    return tiny_kernel


def kernel(x, table, gamma, beta, W, b):
    B, SEQ = x.shape
    tcols = jnp.pad(table.T, ((0, 0), (0, LANES - table.shape[0])))
    params = jnp.concatenate(
        [gamma, beta, W.reshape(-1), b,
         jnp.zeros((32 - 18,), jnp.float32)]).astype(jnp.float32)
    return _build(B, SEQ)(x, tcols, params)


# speedup vs baseline: 1.1827x; 1.1827x over previous
import functools

import jax
import jax.numpy as jnp
from jax import lax
from jax.experimental import pallas as pl
from jax.experimental.pallas import tpu as pltpu
from jax.experimental.pallas import tpu_sc as plsc

NC, NS = 2, 16
NW = NC * NS
LANES = 16


def _rsqrt(v):
    s = v * 0.5 + 0.5
    for _ in range(24):
        s = 0.5 * (s + v / s)
    return 1.0 / s


@functools.lru_cache(maxsize=None)
def _build(B, SEQ):
    rows_per_w = B // NW
    chunk = 128
    nchunk = rows_per_w // chunk
    half = SEQ // 2

    mesh = plsc.VectorSubcoreMesh(
        core_axis_name="c", subcore_axis_name="s",
        num_cores=NC, num_subcores=NS)

    @functools.partial(
        pl.kernel,
        out_type=jax.ShapeDtypeStruct((B, 2), jnp.float32),
        mesh=mesh,
        scratch_types=[
            pltpu.VMEM((chunk * SEQ,), jnp.int32),
            pltpu.VMEM((chunk * SEQ,), jnp.int32),
            pltpu.VMEM((4, LANES), jnp.float32),
            pltpu.VMEM((32,), jnp.float32),
            pltpu.VMEM((112,), jnp.float32),
            pltpu.VMEM((112,), jnp.float32),
            pltpu.VMEM((rows_per_w, 2), jnp.float32),
            pltpu.SemaphoreType.DMA,
            pltpu.SemaphoreType.DMA,
        ],
        compiler_params=pltpu.CompilerParams(use_tc_tiling_on_sc=False,
                                             needs_layout_passes=False),
    )
    def tiny_kernel(x_hbm, tcols_hbm, params_hbm, out_hbm,
                    xv0, xv1, tcols_v, params_v, pair0_v, pair1_v, outv,
                    sem0, sem1):
        wid = lax.axis_index("s") * NC + lax.axis_index("c")
        xbufs, sems = (xv0, xv1), (sem0, sem1)

        def start(ci, buf):
            base = (wid * rows_per_w + ci * chunk) * SEQ
            return pltpu.async_copy(
                x_hbm.at[pl.ds(base, chunk * SEQ)], xbufs[buf], sems[buf])

        cp0 = start(0, 0)

        pltpu.sync_copy(tcols_hbm, tcols_v)
        pltpu.sync_copy(params_hbm, params_v)

        pa = params_v[pl.ds(0, LANES)]
        pb = params_v[pl.ds(LANES, LANES)]

        c = [tcols_v[k] for k in range(4)]
        mu = (c[0] + c[1] + c[2] + c[3]) * 0.25
        d = [ck - mu for ck in c]
        var = (d[0] * d[0] + d[1] * d[1] + d[2] * d[2] + d[3] * d[3]) * 0.25
        r = _rsqrt(var + 1e-5)
        ln = [d[k] * r * pa[k] + pa[4 + k] for k in range(4)]
        inv = 1.0 / SEQ
        t0 = (ln[0] * pa[8] + ln[1] * pa[9]
              + ln[2] * pa[10] + ln[3] * pa[11]
              + pb[0]) * inv
        t1 = (ln[0] * pa[12] + ln[1] * pa[13]
              + ln[2] * pa[14] + ln[3] * pa[15]
              + pb[1]) * inv

        for hi in range(10):
            pair0_v[pl.ds(10 * hi, LANES)] = t0[hi] + t0
            pair1_v[pl.ds(10 * hi, LANES)] = t1[hi] + t1

        iota = lax.iota(jnp.int32, LANES)
        zeros = jnp.zeros((LANES,), jnp.float32)
        zero_i = jnp.zeros((LANES,), jnp.int32)
        one_i = jnp.ones((LANES,), jnp.int32)

        cps = [cp0, None]
        for ci in range(nchunk):
            buf = ci % 2
            cps[buf].wait()
            if ci + 1 < nchunk:
                cps[(ci + 1) % 2] = start(ci + 1, (ci + 1) % 2)
            xv = xbufs[buf]
            for g in range(chunk // LANES):
                rowbase = (g * LANES + iota) * SEQ

                def pbody(l, acc, rowbase=rowbase, xv=xv):
                    a0, a1 = acc
                    i1 = rowbase + l
                    xa = plsc.load_gather(xv, [i1])
                    xb = plsc.load_gather(xv, [i1 + half])
                    q = xa * 10 + xb
                    a0 = a0 + plsc.load_gather(pair0_v, [q])
                    a1 = a1 + plsc.load_gather(pair1_v, [q])
                    return a0, a1

                a0, a1 = lax.fori_loop(0, half, pbody, (zeros, zeros),
                                       unroll=10)
                orow = ci * chunk + g * LANES + iota
                plsc.store_scatter(outv, [orow, zero_i], a0)
                plsc.store_scatter(outv, [orow, one_i], a1)

        pltpu.sync_copy(outv, out_hbm.at[pl.ds(wid * rows_per_w, rows_per_w)])

    return tiny_kernel


def kernel(x, table, gamma, beta, W, b):
    B, SEQ = x.shape
    tcols = jnp.pad(table.T, ((0, 0), (0, LANES - table.shape[0])))
    params = jnp.concatenate(
        [gamma, beta, W.reshape(-1), b,
         jnp.zeros((32 - 18,), jnp.float32)]).astype(jnp.float32)
    return _build(B, SEQ)(x.reshape(-1), tcols, params)


# speedup vs baseline: 1.9346x; 1.6357x over previous
import functools

import jax
import jax.numpy as jnp
from jax import lax
from jax.experimental import pallas as pl
from jax.experimental.pallas import tpu as pltpu
from jax.experimental.pallas import tpu_sc as plsc

NC, NS = 2, 16
NW = NC * NS
LANES = 16


def _rsqrt(v):
    s = v * 0.5 + 0.5
    for _ in range(24):
        s = 0.5 * (s + v / s)
    return 1.0 / s


@functools.lru_cache(maxsize=None)
def _build(B, SEQ):
    rows_per_w = B // NW
    chunk = 128
    nchunk = rows_per_w // chunk
    nfull = SEQ // LANES
    tail = SEQ - nfull * LANES
    tail_c0 = SEQ - LANES

    mesh = plsc.VectorSubcoreMesh(
        core_axis_name="c", subcore_axis_name="s",
        num_cores=NC, num_subcores=NS)

    @functools.partial(
        pl.kernel,
        out_type=jax.ShapeDtypeStruct((2 * B,), jnp.float32),
        mesh=mesh,
        scratch_types=[
            pltpu.VMEM((chunk, SEQ), jnp.int32),
            pltpu.VMEM((chunk, SEQ), jnp.int32),
            pltpu.VMEM((4, LANES), jnp.float32),
            pltpu.VMEM((32,), jnp.float32),
            pltpu.VMEM((112,), jnp.float32),
            pltpu.VMEM((112,), jnp.float32),
            pltpu.VMEM((LANES,), jnp.float32),
            pltpu.VMEM((LANES,), jnp.float32),
            pltpu.VMEM((2 * rows_per_w,), jnp.float32),
            pltpu.SemaphoreType.DMA,
            pltpu.SemaphoreType.DMA,
        ],
        compiler_params=pltpu.CompilerParams(use_tc_tiling_on_sc=True,
                                             needs_layout_passes=False),
    )
    def tiny_kernel(x_hbm, tcols_hbm, params_hbm, out_hbm,
                    xv0, xv1, tcols_v, params_v, pair0_v, pair1_v,
                    t0_v, t1_v, outv, sem0, sem1):
        wid = lax.axis_index("s") * NC + lax.axis_index("c")
        xbufs, sems = (xv0, xv1), (sem0, sem1)

        def start(ci, buf):
            base = wid * rows_per_w + ci * chunk
            return pltpu.async_copy(
                x_hbm.at[pl.ds(base, chunk), :], xbufs[buf], sems[buf])

        cp0 = start(0, 0)

        pltpu.sync_copy(tcols_hbm, tcols_v)
        pltpu.sync_copy(params_hbm, params_v)

        pa = params_v[pl.ds(0, LANES)]
        pb = params_v[pl.ds(LANES, LANES)]

        c = [tcols_v[k] for k in range(4)]
        mu = (c[0] + c[1] + c[2] + c[3]) * 0.25
        d = [ck - mu for ck in c]
        var = (d[0] * d[0] + d[1] * d[1] + d[2] * d[2] + d[3] * d[3]) * 0.25
        r = _rsqrt(var + 1e-5)
        ln = [d[k] * r * pa[k] + pa[4 + k] for k in range(4)]
        inv = 1.0 / SEQ
        t0 = (ln[0] * pa[8] + ln[1] * pa[9]
              + ln[2] * pa[10] + ln[3] * pa[11]
              + pb[0]) * inv
        t1 = (ln[0] * pa[12] + ln[1] * pa[13]
              + ln[2] * pa[14] + ln[3] * pa[15]
              + pb[1]) * inv
        t0_v[...] = t0
        t1_v[...] = t1

        for hi in range(10):
            pair0_v[pl.ds(10 * hi, LANES)] = t0[hi] + t0
            pair1_v[pl.ds(10 * hi, LANES)] = t1[hi] + t1

        iota = lax.iota(jnp.int32, LANES)
        zerof = jnp.zeros((LANES,), jnp.float32)
        tmask = iota >= (LANES - tail)

        cps = [cp0, None]
        for ci in range(nchunk):
            buf = ci % 2
            cps[buf].wait()
            if ci + 1 < nchunk:
                cps[(ci + 1) % 2] = start(ci + 1, (ci + 1) % 2)
            xv = xbufs[buf]

            def gbody(g, _, xv=xv, ci=ci):
                ov0, ov1 = zerof, zerof
                for s in range(LANES):
                    row = g * LANES + s
                    v = [xv[row, pl.ds(16 * m, LANES)] for m in range(nfull)]
                    tl = xv[row, pl.ds(tail_c0, LANES)]
                    a0, a1 = zerof, zerof
                    for m in range(nfull // 2):
                        q = v[2 * m] * 10 + v[2 * m + 1]
                        a0 = a0 + plsc.load_gather(pair0_v, [q])
                        a1 = a1 + plsc.load_gather(pair1_v, [q])
                    a0 = a0 + jnp.where(tmask, plsc.load_gather(t0_v, [tl]),
                                        0.0)
                    a1 = a1 + jnp.where(tmask, plsc.load_gather(t1_v, [tl]),
                                        0.0)
                    s0 = jnp.sum(a0)
                    s1 = jnp.sum(a1)
                    ov0 = jnp.where(iota == s, s0, ov0)
                    ov1 = jnp.where(iota == s, s1, ov1)
                base = ci * chunk + g * LANES
                outv[pl.ds(base, LANES)] = ov0
                outv[pl.ds(rows_per_w + base, LANES)] = ov1
                return 0

            lax.fori_loop(0, chunk // LANES, gbody, 0)

        pltpu.sync_copy(outv.at[pl.ds(0, rows_per_w)],
                        out_hbm.at[pl.ds(wid * rows_per_w, rows_per_w)])
        pltpu.sync_copy(outv.at[pl.ds(rows_per_w, rows_per_w)],
                        out_hbm.at[pl.ds(B + wid * rows_per_w, rows_per_w)])

    return tiny_kernel


def kernel(x, table, gamma, beta, W, b):
    B, SEQ = x.shape
    tcols = jnp.pad(table.T, ((0, 0), (0, LANES - table.shape[0])))
    params = jnp.concatenate(
        [gamma, beta, W.reshape(-1), b,
         jnp.zeros((32 - 18,), jnp.float32)]).astype(jnp.float32)
    flat = _build(B, SEQ)(x, tcols, params)
    return flat.reshape(2, B).T


# speedup vs baseline: 2.1388x; 1.1055x over previous
import functools

import jax
import jax.numpy as jnp
from jax import lax
from jax.experimental import pallas as pl
from jax.experimental.pallas import tpu as pltpu
from jax.experimental.pallas import tpu_sc as plsc

NC, NS = 2, 16
NW = NC * NS
LANES = 16


def _rsqrt(v):
    s = v * 0.5 + 0.5
    for _ in range(24):
        s = 0.5 * (s + v / s)
    return 1.0 / s


@functools.lru_cache(maxsize=None)
def _build(B, SEQ):
    rows_per_w = B // NW
    chunk = 128
    nchunk = rows_per_w // chunk
    nfull = SEQ // LANES
    tail = SEQ - nfull * LANES
    tail_c0 = SEQ - LANES

    mesh = plsc.VectorSubcoreMesh(
        core_axis_name="c", subcore_axis_name="s",
        num_cores=NC, num_subcores=NS)

    @functools.partial(
        pl.kernel,
        out_type=jax.ShapeDtypeStruct((2 * B,), jnp.float32),
        mesh=mesh,
        scratch_types=[
            pltpu.VMEM((chunk, SEQ), jnp.int32),
            pltpu.VMEM((chunk, SEQ), jnp.int32),
            pltpu.VMEM((4, LANES), jnp.float32),
            pltpu.VMEM((32,), jnp.float32),
            pltpu.VMEM((112,), jnp.float32),
            pltpu.VMEM((112,), jnp.float32),
            pltpu.VMEM((LANES,), jnp.float32),
            pltpu.VMEM((LANES,), jnp.float32),
            pltpu.VMEM((2 * rows_per_w,), jnp.float32),
            pltpu.SemaphoreType.DMA,
            pltpu.SemaphoreType.DMA,
        ],
        compiler_params=pltpu.CompilerParams(use_tc_tiling_on_sc=True,
                                             needs_layout_passes=False),
    )
    def tiny_kernel(x_hbm, tcols_hbm, params_hbm, out_hbm,
                    xv0, xv1, tcols_v, params_v, pair0_v, pair1_v,
                    t0_v, t1_v, outv, sem0, sem1):
        wid = lax.axis_index("s") * NC + lax.axis_index("c")
        xbufs, sems = (xv0, xv1), (sem0, sem1)

        def start(ci, buf):
            base = wid * rows_per_w + ci * chunk
            return pltpu.async_copy(
                x_hbm.at[pl.ds(base, chunk), :], xbufs[buf], sems[buf])

        cp0 = start(0, 0)

        pltpu.sync_copy(tcols_hbm, tcols_v)
        pltpu.sync_copy(params_hbm, params_v)

        pa = params_v[pl.ds(0, LANES)]
        pb = params_v[pl.ds(LANES, LANES)]

        c = [tcols_v[k] for k in range(4)]
        mu = (c[0] + c[1] + c[2] + c[3]) * 0.25
        d = [ck - mu for ck in c]
        var = (d[0] * d[0] + d[1] * d[1] + d[2] * d[2] + d[3] * d[3]) * 0.25
        r = _rsqrt(var + 1e-5)
        ln = [d[k] * r * pa[k] + pa[4 + k] for k in range(4)]
        inv = 1.0 / SEQ
        t0 = (ln[0] * pa[8] + ln[1] * pa[9]
              + ln[2] * pa[10] + ln[3] * pa[11]
              + pb[0]) * inv
        t1 = (ln[0] * pa[12] + ln[1] * pa[13]
              + ln[2] * pa[14] + ln[3] * pa[15]
              + pb[1]) * inv
        t0_v[...] = t0
        t1_v[...] = t1

        for hi in range(10):
            pair0_v[pl.ds(10 * hi, LANES)] = t0[hi] + t0
            pair1_v[pl.ds(10 * hi, LANES)] = t1[hi] + t1

        iota = lax.iota(jnp.int32, LANES)
        zerof = jnp.zeros((LANES,), jnp.float32)
        tmask = iota >= (LANES - tail)

        cps = [cp0, None]
        for ci in range(nchunk):
            buf = ci % 2
            cps[buf].wait()
            if ci + 1 < nchunk:
                cps[(ci + 1) % 2] = start(ci + 1, (ci + 1) % 2)
            xv = xbufs[buf]

            def gbody(g, _, xv=xv, ci=ci):
                def sbody(s, ov, xv=xv, g=g):
                    ov0, ov1 = ov
                    row = g * LANES + s
                    v = [xv[row, pl.ds(16 * m, LANES)] for m in range(nfull)]
                    tl = xv[row, pl.ds(tail_c0, LANES)]
                    a0, a1 = zerof, zerof
                    for m in range(nfull // 2):
                        q = v[2 * m] * 10 + v[2 * m + 1]
                        a0 = a0 + plsc.load_gather(pair0_v, [q])
                        a1 = a1 + plsc.load_gather(pair1_v, [q])
                    a0 = a0 + jnp.where(tmask, plsc.load_gather(t0_v, [tl]),
                                        0.0)
                    a1 = a1 + jnp.where(tmask, plsc.load_gather(t1_v, [tl]),
                                        0.0)
                    s0 = jnp.sum(a0)
                    s1 = jnp.sum(a1)
                    return (jnp.where(iota == s, s0, ov0),
                            jnp.where(iota == s, s1, ov1))

                ov0, ov1 = lax.fori_loop(0, LANES, sbody, (zerof, zerof),
                                         unroll=2)
                base = ci * chunk + g * LANES
                outv[pl.ds(base, LANES)] = ov0
                outv[pl.ds(rows_per_w + base, LANES)] = ov1
                return 0

            lax.fori_loop(0, chunk // LANES, gbody, 0)

        pltpu.sync_copy(outv.at[pl.ds(0, rows_per_w)],
                        out_hbm.at[pl.ds(wid * rows_per_w, rows_per_w)])
        pltpu.sync_copy(outv.at[pl.ds(rows_per_w, rows_per_w)],
                        out_hbm.at[pl.ds(B + wid * rows_per_w, rows_per_w)])

    return tiny_kernel


def kernel(x, table, gamma, beta, W, b):
    B, SEQ = x.shape
    tcols = jnp.pad(table.T, ((0, 0), (0, LANES - table.shape[0])))
    params = jnp.concatenate(
        [gamma, beta, W.reshape(-1), b,
         jnp.zeros((32 - 18,), jnp.float32)]).astype(jnp.float32)
    flat = _build(B, SEQ)(x, tcols, params)
    return flat.reshape(2, B).T
